# baseline (speedup 1.0000x reference)
import functools

import jax
import jax.numpy as jnp
from jax import lax
from jax.experimental import pallas as pl
from jax.experimental.pallas import tpu as pltpu
from jax.experimental.pallas import tpu_sc as plsc

L = 16

_LN2 = 0.6931471805599453


def _vlog(x):
    xi = lax.bitcast_convert_type(x, jnp.int32)
    e = ((xi >> 23) & 0xFF) - 127
    mi = (xi & 0x7FFFFF) | 0x3F800000
    m = lax.bitcast_convert_type(mi, jnp.float32)
    t = (m - 1.0) / (m + 1.0)
    t2 = t * t
    log_m = t * (2.0 + t2 * (2.0 / 3.0 + t2 * (2.0 / 5.0 + t2 * (2.0 / 7.0))))
    return e.astype(jnp.float32) * _LN2 + log_m


def _lane_shuffle(v, perm):
    return v.at[perm].get(mode="promise_in_bounds")


def _lane_max(v):
    lanes = lax.iota(jnp.int32, L)
    for sh in (8, 4, 2, 1):
        v = jnp.maximum(v, _lane_shuffle(v, lanes ^ sh))
    return v


def _lane_sum(v):
    lanes = lax.iota(jnp.int32, L)
    for sh in (8, 4, 2, 1):
        v = v + _lane_shuffle(v, lanes ^ sh)
    return v


def _make_sc_kernel(B, D, P, K):
    NC, NS = 2, 16
    NW = NC * NS
    assert B % NW == 0 and P == 4 * L and D % L == 0
    BPW = B // NW
    mesh = plsc.VectorSubcoreMesh(
        core_axis_name="c", subcore_axis_name="s",
        num_cores=NC, num_subcores=NS)

    @functools.partial(
        pl.kernel,
        out_type=jax.ShapeDtypeStruct((B,), jnp.float32),
        mesh=mesh,
        compiler_params=pltpu.CompilerParams(use_tc_tiling_on_sc=False),
        scratch_types=[
            pltpu.VMEM((BPW, D), jnp.int32),
            pltpu.VMEM((D, P // 2), jnp.int32),
            pltpu.VMEM((D, P // 2), jnp.int32),
            pltpu.VMEM((D, P // 2), jnp.int32),
            pltpu.VMEM((D, P // 2), jnp.int32),
            pltpu.VMEM((P,), jnp.float32),
            pltpu.VMEM((BPW,), jnp.float32),
            pltpu.SemaphoreType.DMA,
            pltpu.SemaphoreType.DMA,
            pltpu.SemaphoreType.DMA,
            pltpu.SemaphoreType.DMA,
        ],
    )
    def body(y_hbm, alpha_hbm, table_hbm, out_hbm,
             idx_v, rows0, rows1, rows2, rows3, alpha_v, out_v,
             sem0, sem1, sem2, sem3):
        wid = lax.axis_index("s") * NC + lax.axis_index("c")
        base = wid * BPW
        bufs = (rows0, rows1, rows2, rows3)
        sems = (sem0, sem1, sem2, sem3)

        pltpu.sync_copy(y_hbm.at[pl.ds(base, BPW)], idx_v)
        pltpu.sync_copy(alpha_hbm, alpha_v)

        offs = [(lax.iota(jnp.int32, L) + j * L) * K for j in range(D // L)]

        def add_offs(b, carry):
            for j in range(D // L):
                sl = pl.ds(j * L, L)
                idx_v[b, sl] = idx_v[b, sl] + offs[j]
            return carry

        lax.fori_loop(0, BPW, add_offs, 0)

        alpha = [alpha_v[pl.ds(s, L)] for s in (0, 2 * L, L, 3 * L)]

        def start(b, j):
            pltpu.async_copy(table_hbm.at[idx_v.at[b]], bufs[j], sems[j])

        def wait(j):
            pltpu.make_async_copy(
                table_hbm.at[pl.ds(0, D)], bufs[j], sems[j]).wait()

        for j in range(4):
            start(j, j)

        def acc_pair(bufa, bufb):
            def wide(buf, d, half):
                xi = buf[d, pl.ds(half * L, L)]
                lo = lax.bitcast_convert_type(xi << 16, jnp.float32)
                hi = lax.bitcast_convert_type(xi & jnp.int32(-65536),
                                              jnp.float32)
                return lo, hi

            def acc_step(t, carry):
                acc = list(carry)
                for u in range(4):
                    d = t * 4 + u
                    for h in range(2):
                        ea, oa = wide(bufa, d, h)
                        eb, ob = wide(bufb, d, h)
                        acc[2 * h] = acc[2 * h] + ea
                        acc[2 * h + 1] = acc[2 * h + 1] + oa
                        acc[4 + 2 * h] = acc[4 + 2 * h] + eb
                        acc[4 + 2 * h + 1] = acc[4 + 2 * h + 1] + ob
                return tuple(acc)

            z = jnp.zeros((L,), jnp.float32)
            return lax.fori_loop(0, D // 4, acc_step, (z,) * 8)

        def finish(b, acc, res_v):
            a = [acc[c] + alpha[c] for c in range(4)]
            m = _lane_max(jnp.maximum(jnp.maximum(a[0], a[1]),
                                      jnp.maximum(a[2], a[3])))
            s = (jnp.exp(a[0] - m) + jnp.exp(a[1] - m)
                 + jnp.exp(a[2] - m) + jnp.exp(a[3] - m))
            lse = _vlog(_lane_sum(s)) + m
            lane = b % L
            res_v = jnp.where(lax.iota(jnp.int32, L) == lane, lse, res_v)

            @pl.when(lane == L - 1)
            def _():
                out_v[pl.ds(b - (L - 1), L)] = res_v

            return res_v

        def main_step(q, res_v):
            b0 = 4 * q
            for jp in range(2):
                ja, jb = 2 * jp, 2 * jp + 1
                wait(ja)
                wait(jb)
                acc = acc_pair(bufs[ja], bufs[jb])
                for j, off in ((ja, 0), (jb, 1)):
                    b = b0 + jp * 2 + off
                    res_v = finish(b, acc[off * 4:off * 4 + 4], res_v)

                    @pl.when(b + 4 < BPW)
                    def _():
                        start(b + 4, j)
            return res_v

        lax.fori_loop(0, BPW // 4, main_step, jnp.zeros((L,), jnp.float32))
        pltpu.sync_copy(out_v, out_hbm.at[pl.ds(base, BPW)])

    return body


def _make_tc_pack(D, P, K):
    W = P // 2

    def body(x_ref, o_ref):
        x = x_ref[0]
        a16 = lax.bitcast_convert_type(
            x[:W, :].astype(jnp.bfloat16), jnp.uint16)
        b16 = lax.bitcast_convert_type(
            x[W:, :].astype(jnp.bfloat16), jnp.uint16)
        w = lax.bitcast_convert_type(
            a16.astype(jnp.uint32) | (b16.astype(jnp.uint32) << 16),
            jnp.int32)
        o_ref[...] = jnp.transpose(w, (1, 0))

    return pl.pallas_call(
        body,
        grid=(D,),
        in_specs=[pl.BlockSpec((1, P, K), lambda d: (d, 0, 0))],
        out_specs=pl.BlockSpec((K, W), lambda d: (d, 0)),
        out_shape=jax.ShapeDtypeStruct((D * K, W), jnp.int32),
    )


def kernel(y, log_alpha, log_cp_cores):
    D, P, K = log_cp_cores.shape
    B = y.shape[0]
    table = _make_tc_pack(D, P, K)(log_cp_cores)
    sc = _make_sc_kernel(B, D, P, K)
    return sc(y.astype(jnp.int32), log_alpha, table)


# speedup vs baseline: 1.5769x; 1.5769x over previous
import functools

import jax
import jax.numpy as jnp
from jax import lax
from jax.experimental import pallas as pl
from jax.experimental.pallas import tpu as pltpu
from jax.experimental.pallas import tpu_sc as plsc

L = 16

_LN2 = 0.6931471805599453


def _vlog(x):
    xi = lax.bitcast_convert_type(x, jnp.int32)
    e = ((xi >> 23) & 0xFF) - 127
    mi = (xi & 0x7FFFFF) | 0x3F800000
    m = lax.bitcast_convert_type(mi, jnp.float32)
    t = (m - 1.0) / (m + 1.0)
    t2 = t * t
    log_m = t * (2.0 + t2 * (2.0 / 3.0 + t2 * (2.0 / 5.0 + t2 * (2.0 / 7.0))))
    return e.astype(jnp.float32) * _LN2 + log_m


def _lane_shuffle(v, perm):
    return v.at[perm].get(mode="promise_in_bounds")


def _lane_max(v):
    lanes = lax.iota(jnp.int32, L)
    for sh in (8, 4, 2, 1):
        v = jnp.maximum(v, _lane_shuffle(v, lanes ^ sh))
    return v


def _lane_sum(v):
    lanes = lax.iota(jnp.int32, L)
    for sh in (8, 4, 2, 1):
        v = v + _lane_shuffle(v, lanes ^ sh)
    return v


def _make_sc_kernel(B, D, P, K):
    NC, NS = 2, 16
    NW = NC * NS
    assert B % NW == 0 and P == 4 * L and D % L == 0
    BPW = B // NW
    mesh = plsc.VectorSubcoreMesh(
        core_axis_name="c", subcore_axis_name="s",
        num_cores=NC, num_subcores=NS)

    @functools.partial(
        pl.kernel,
        out_type=jax.ShapeDtypeStruct((B,), jnp.float32),
        mesh=mesh,
        compiler_params=pltpu.CompilerParams(use_tc_tiling_on_sc=False),
        scratch_types=[
            pltpu.VMEM((BPW, D), jnp.int32),
            pltpu.VMEM((D, P // 2), jnp.int32),
            pltpu.VMEM((D, P // 2), jnp.int32),
            pltpu.VMEM((D, P // 2), jnp.int32),
            pltpu.VMEM((D, P // 2), jnp.int32),
            pltpu.VMEM((P,), jnp.float32),
            pltpu.VMEM((BPW,), jnp.float32),
            pltpu.SemaphoreType.DMA,
            pltpu.SemaphoreType.DMA,
            pltpu.SemaphoreType.DMA,
            pltpu.SemaphoreType.DMA,
        ],
    )
    def body(y_hbm, alpha_hbm, table_hbm, out_hbm,
             idx_v, rows0, rows1, rows2, rows3, alpha_v, out_v,
             sem0, sem1, sem2, sem3):
        wid = lax.axis_index("s") * NC + lax.axis_index("c")
        base = wid * BPW
        bufs = (rows0, rows1, rows2, rows3)
        sems = (sem0, sem1, sem2, sem3)

        pltpu.sync_copy(y_hbm.at[pl.ds(base, BPW)], idx_v)
        pltpu.sync_copy(alpha_hbm, alpha_v)

        offs = [(lax.iota(jnp.int32, L) + j * L) * K for j in range(D // L)]

        def add_offs(b, carry):
            for j in range(D // L):
                sl = pl.ds(j * L, L)
                idx_v[b, sl] = idx_v[b, sl] + offs[j]
            return carry

        lax.fori_loop(0, BPW, add_offs, 0)

        alpha = [alpha_v[pl.ds(s, L)] for s in (0, 2 * L, L, 3 * L)]

        def start(b, j):
            pltpu.async_copy(table_hbm.at[idx_v.at[b]], bufs[j], sems[j])

        def wait(j):
            pltpu.make_async_copy(
                table_hbm.at[pl.ds(0, D)], bufs[j], sems[j]).wait()

        for j in range(4):
            start(j, j)

        def acc_pair(bufa, bufb):
            def wide(buf, d, half):
                xi = buf[d, pl.ds(half * L, L)]
                lo = lax.bitcast_convert_type(xi << 16, jnp.float32)
                hi = lax.bitcast_convert_type(xi & jnp.int32(-65536),
                                              jnp.float32)
                return lo, hi

            def acc_step(t, carry):
                acc = list(carry)
                for u in range(4):
                    d = t * 4 + u
                    for h in range(2):
                        ea, oa = wide(bufa, d, h)
                        eb, ob = wide(bufb, d, h)
                        acc[2 * h] = acc[2 * h] + ea
                        acc[2 * h + 1] = acc[2 * h + 1] + oa
                        acc[4 + 2 * h] = acc[4 + 2 * h] + eb
                        acc[4 + 2 * h + 1] = acc[4 + 2 * h + 1] + ob
                return tuple(acc)

            z = jnp.zeros((L,), jnp.float32)
            return lax.fori_loop(0, D // 4, acc_step, (z,) * 8)

        def finish(b, acc, res_v):
            a = [acc[c] + alpha[c] for c in range(4)]
            m = _lane_max(jnp.maximum(jnp.maximum(a[0], a[1]),
                                      jnp.maximum(a[2], a[3])))
            s = (jnp.exp(a[0] - m) + jnp.exp(a[1] - m)
                 + jnp.exp(a[2] - m) + jnp.exp(a[3] - m))
            lse = _vlog(_lane_sum(s)) + m
            lane = b % L
            res_v = jnp.where(lax.iota(jnp.int32, L) == lane, lse, res_v)

            @pl.when(lane == L - 1)
            def _():
                out_v[pl.ds(b - (L - 1), L)] = res_v

            return res_v

        def main_step(q, res_v):
            b0 = 4 * q
            for jp in range(2):
                ja, jb = 2 * jp, 2 * jp + 1
                wait(ja)
                wait(jb)
                acc = acc_pair(bufs[ja], bufs[jb])
                for j, off in ((ja, 0), (jb, 1)):
                    b = b0 + jp * 2 + off
                    res_v = finish(b, acc[off * 4:off * 4 + 4], res_v)

                    @pl.when(b + 4 < BPW)
                    def _():
                        start(b + 4, j)
            return res_v

        lax.fori_loop(0, BPW // 4, main_step, jnp.zeros((L,), jnp.float32))
        pltpu.sync_copy(out_v, out_hbm.at[pl.ds(base, BPW)])

    return body


def kernel(y, log_alpha, log_cp_cores):
    D, P, K = log_cp_cores.shape
    B = y.shape[0]
    a16 = lax.bitcast_convert_type(
        log_cp_cores[:, :P // 2, :].astype(jnp.bfloat16), jnp.uint16)
    b16 = lax.bitcast_convert_type(
        log_cp_cores[:, P // 2:, :].astype(jnp.bfloat16), jnp.uint16)
    packed = lax.bitcast_convert_type(
        a16.astype(jnp.uint32) | (b16.astype(jnp.uint32) << 16), jnp.int32)
    table = lax.reshape(packed, (D * K, P // 2), dimensions=(0, 2, 1))
    sc = _make_sc_kernel(B, D, P, K)
    return sc(y.astype(jnp.int32), log_alpha, table)


# speedup vs baseline: 1.5781x; 1.0007x over previous
import functools

import jax
import jax.numpy as jnp
from jax import lax
from jax.experimental import pallas as pl
from jax.experimental.pallas import tpu as pltpu
from jax.experimental.pallas import tpu_sc as plsc

L = 16

_LN2 = 0.6931471805599453


def _vlog(x):
    xi = lax.bitcast_convert_type(x, jnp.int32)
    e = ((xi >> 23) & 0xFF) - 127
    mi = (xi & 0x7FFFFF) | 0x3F800000
    m = lax.bitcast_convert_type(mi, jnp.float32)
    t = (m - 1.0) / (m + 1.0)
    t2 = t * t
    log_m = t * (2.0 + t2 * (2.0 / 3.0 + t2 * (2.0 / 5.0 + t2 * (2.0 / 7.0))))
    return e.astype(jnp.float32) * _LN2 + log_m


def _lane_shuffle(v, perm):
    return v.at[perm].get(mode="promise_in_bounds")


def _lane_max(v):
    lanes = lax.iota(jnp.int32, L)
    for sh in (8, 4, 2, 1):
        v = jnp.maximum(v, _lane_shuffle(v, lanes ^ sh))
    return v


def _lane_sum(v):
    lanes = lax.iota(jnp.int32, L)
    for sh in (8, 4, 2, 1):
        v = v + _lane_shuffle(v, lanes ^ sh)
    return v


def _make_sc_kernel(B, D, P, K):
    NC, NS = 2, 16
    NW = NC * NS
    assert B % NW == 0 and P == 4 * L and D % L == 0
    BPW = B // NW
    mesh = plsc.VectorSubcoreMesh(
        core_axis_name="c", subcore_axis_name="s",
        num_cores=NC, num_subcores=NS)

    @functools.partial(
        pl.kernel,
        out_type=jax.ShapeDtypeStruct((B,), jnp.float32),
        mesh=mesh,
        compiler_params=pltpu.CompilerParams(use_tc_tiling_on_sc=False),
        scratch_types=[
            pltpu.VMEM((BPW, D), jnp.int32),
            pltpu.VMEM((D, P // 2), jnp.int32),
            pltpu.VMEM((D, P // 2), jnp.int32),
            pltpu.VMEM((D, P // 2), jnp.int32),
            pltpu.VMEM((D, P // 2), jnp.int32),
            pltpu.VMEM((P,), jnp.float32),
            pltpu.VMEM((BPW,), jnp.float32),
            pltpu.SemaphoreType.DMA,
            pltpu.SemaphoreType.DMA,
            pltpu.SemaphoreType.DMA,
            pltpu.SemaphoreType.DMA,
        ],
    )
    def body(y_hbm, alpha_hbm, table_hbm, out_hbm,
             idx_v, rows0, rows1, rows2, rows3, alpha_v, out_v,
             sem0, sem1, sem2, sem3):
        wid = lax.axis_index("s") * NC + lax.axis_index("c")
        base = wid * BPW
        bufs = (rows0, rows1, rows2, rows3)
        sems = (sem0, sem1, sem2, sem3)

        pltpu.sync_copy(y_hbm.at[pl.ds(base, BPW)], idx_v)
        pltpu.sync_copy(alpha_hbm, alpha_v)

        offs = [(lax.iota(jnp.int32, L) + j * L) * K for j in range(D // L)]

        def add_offs(b, carry):
            for j in range(D // L):
                sl = pl.ds(j * L, L)
                idx_v[b, sl] = idx_v[b, sl] + offs[j]
            return carry

        lax.fori_loop(0, BPW, add_offs, 0)

        alpha = [alpha_v[pl.ds(s, L)] for s in (0, 2 * L, L, 3 * L)]

        def start(b, j):
            pltpu.async_copy(table_hbm.at[idx_v.at[b]], bufs[j], sems[j])

        def wait(j):
            pltpu.make_async_copy(
                table_hbm.at[pl.ds(0, D)], bufs[j], sems[j]).wait()

        for j in range(4):
            start(j, j)

        def acc_pair(bufa, bufb):
            def wide(buf, d, half):
                xi = buf[d, pl.ds(half * L, L)]
                lo = lax.bitcast_convert_type(xi << 16, jnp.float32)
                hi = lax.bitcast_convert_type(xi & jnp.int32(-65536),
                                              jnp.float32)
                return lo, hi

            def acc_step(t, carry):
                acc = list(carry)
                for u in range(4):
                    d = t * 4 + u
                    for h in range(2):
                        ea, oa = wide(bufa, d, h)
                        eb, ob = wide(bufb, d, h)
                        acc[2 * h] = acc[2 * h] + ea
                        acc[2 * h + 1] = acc[2 * h + 1] + oa
                        acc[4 + 2 * h] = acc[4 + 2 * h] + eb
                        acc[4 + 2 * h + 1] = acc[4 + 2 * h + 1] + ob
                return tuple(acc)

            z = jnp.zeros((L,), jnp.float32)
            return plsc.parallel_loop(
                0, D // 4, carry=(z,) * 8, unroll=2)(acc_step)

        def finish(b, acc, res_v):
            a = [acc[c] + alpha[c] for c in range(4)]
            m = _lane_max(jnp.maximum(jnp.maximum(a[0], a[1]),
                                      jnp.maximum(a[2], a[3])))
            s = (jnp.exp(a[0] - m) + jnp.exp(a[1] - m)
                 + jnp.exp(a[2] - m) + jnp.exp(a[3] - m))
            lse = _vlog(_lane_sum(s)) + m
            lane = b % L
            res_v = jnp.where(lax.iota(jnp.int32, L) == lane, lse, res_v)

            @pl.when(lane == L - 1)
            def _():
                out_v[pl.ds(b - (L - 1), L)] = res_v

            return res_v

        def main_step(q, res_v):
            b0 = 4 * q
            for jp in range(2):
                ja, jb = 2 * jp, 2 * jp + 1
                wait(ja)
                wait(jb)
                acc = acc_pair(bufs[ja], bufs[jb])
                for j, off in ((ja, 0), (jb, 1)):
                    b = b0 + jp * 2 + off
                    res_v = finish(b, acc[off * 4:off * 4 + 4], res_v)

                    @pl.when(b + 4 < BPW)
                    def _():
                        start(b + 4, j)
            return res_v

        lax.fori_loop(0, BPW // 4, main_step, jnp.zeros((L,), jnp.float32))
        pltpu.sync_copy(out_v, out_hbm.at[pl.ds(base, BPW)])

    return body


def kernel(y, log_alpha, log_cp_cores):
    D, P, K = log_cp_cores.shape
    B = y.shape[0]
    a16 = lax.bitcast_convert_type(
        log_cp_cores[:, :P // 2, :].astype(jnp.bfloat16), jnp.uint16)
    b16 = lax.bitcast_convert_type(
        log_cp_cores[:, P // 2:, :].astype(jnp.bfloat16), jnp.uint16)
    packed = lax.bitcast_convert_type(
        a16.astype(jnp.uint32) | (b16.astype(jnp.uint32) << 16), jnp.int32)
    table = lax.reshape(packed, (D * K, P // 2), dimensions=(0, 2, 1))
    sc = _make_sc_kernel(B, D, P, K)
    return sc(y.astype(jnp.int32), log_alpha, table)
